# baseline (speedup 1.0000x reference)
import functools
import math

import jax
import jax.numpy as jnp
from jax import lax
from jax.experimental import pallas as pl
from jax.experimental.pallas import tpu as pltpu
import jax.experimental.pallas.tpu_sc as plsc

D_MODEL = 64
SCALE = math.sqrt(D_MODEL)
VOCAB = 1000000
NC = 2
NS = 16
NW = NC * NS
LANES = 16

def _mesh():
    return plsc.VectorSubcoreMesh(core_axis_name="c", subcore_axis_name="s",
                                  num_cores=NC, num_subcores=NS)


def _gather_kernel():
    per_w = 6400 // NW

    @functools.partial(
        pl.kernel,
        out_type=jax.ShapeDtypeStruct((200, 8, 32, 8, 128), jnp.float32),
        mesh=_mesh(),
        scratch_types=[
            pltpu.VMEM((200, 128), jnp.int32),
            pltpu.VMEM((128, D_MODEL), jnp.float32),
            pltpu.VMEM((128, D_MODEL), jnp.float32),
            pltpu.VMEM((128, D_MODEL), jnp.float32),
            pltpu.VMEM((128, D_MODEL), jnp.float32),
            pltpu.VMEM((D_MODEL, 128), jnp.float32),
            pltpu.VMEM((D_MODEL, 128), jnp.float32),
            pltpu.SemaphoreType.DMA,
            pltpu.SemaphoreType.DMA,
            pltpu.SemaphoreType.DMA,
            pltpu.SemaphoreType.DMA,
            pltpu.SemaphoreType.DMA,
            pltpu.SemaphoreType.DMA,
            pltpu.SemaphoreType.DMA,
        ],
        compiler_params=pltpu.CompilerParams(use_tc_tiling_on_sc=False,
                                             needs_layout_passes=False),
    )
    def gather(xf_hbm, tab_hbm, out_hbm, idxa, gb0, gb1, gb2, gb3, ob0, ob1,
               isem, gsem0, gsem1, gsem2, gsem3, osem0, osem1):
        wid = lax.axis_index("s") * NC + lax.axis_index("c")
        rows8 = [lax.iota(jnp.int32, LANES) + (16 * c) for c in range(8)]
        gb = (gb0, gb1, gb2, gb3)
        ob = (ob0, ob1)
        gsem = (gsem0, gsem1, gsem2, gsem3)
        osem = (osem0, osem1)

        pltpu.async_copy(xf_hbm.at[pl.ds(wid * per_w, per_w)], idxa, isem)
        pltpu.make_async_copy(xf_hbm.at[pl.ds(0, per_w)], idxa, isem).wait()

        def start_g(i, b):
            pltpu.async_copy(tab_hbm.at[idxa.at[i]], gb[b], gsem[b])

        def wait_g(b):
            pltpu.make_async_copy(tab_hbm.at[idxa.at[0]], gb[b],
                                  gsem[b]).wait()

        def start_out(i, b):
            blk = wid * per_w + i
            s = blk // 32
            bt = blk % 32
            for dt in range(8):
                pltpu.async_copy(ob[b].at[pl.ds(8 * dt, 8)],
                                 out_hbm.at[s, dt, bt], osem[b])

        def wait_out(b):
            for _ in range(8):
                pltpu.make_async_copy(ob[b].at[pl.ds(0, 8)],
                                      out_hbm.at[0, 0, 0], osem[b]).wait()

        def transpose(g, o):
            @plsc.parallel_loop(0, D_MODEL, 1, unroll=4)
            def _col(col):
                cols = jnp.full((LANES,), col, jnp.int32)
                for c in range(8):
                    v = plsc.load_gather(gb[g], [rows8[c], cols])
                    ob[o][col, pl.ds(16 * c, 16)] = v * SCALE

        n = per_w
        for i in range(4):
            start_g(i, i)
        for i in range(4):
            wait_g(i)
            if i >= 2:
                wait_out(i % 2)
            transpose(i, i % 2)
            start_out(i, i % 2)
            start_g(i + 4, i)

        @pl.loop(4, n - 4, step=4)
        def _group(g):
            for b in range(4):
                i = g + b
                wait_g(b)
                wait_out(b % 2)
                transpose(b, b % 2)
                start_out(i, b % 2)
                start_g(i + 4, b)

        for i in (n - 4, n - 3, n - 2, n - 1):
            b = i % 4
            wait_g(b)
            wait_out(b % 2)
            transpose(b, b % 2)
            start_out(i, b % 2)
        for b in range(2):
            wait_out(b)

    return gather


def kernel(x, table):
    b, s = x.shape
    xf = x.T.reshape((b * s) // 128, 128)
    out5 = _gather_kernel()(xf, table)
    return jnp.transpose(out5, (2, 4, 0, 1, 3)).reshape(b, s, D_MODEL)
